# baseline (speedup 1.0000x reference)
import jax
import jax.numpy as jnp
from jax import lax
from jax.experimental import pallas as pl
from jax.experimental.pallas import tpu as pltpu
from jax.experimental.pallas import tpu_sc as plsc

_PATHS = 64
_T = 32768
_D = 768
_NC = 2
_NS = 16
_NW = _NC * _NS
_ROWS_W = _T // _NW
_CH = 32
_NCHUNK = _ROWS_W // _CH
_NBUF = 4


def _dispatch_body(x_hbm, out_hbm, *scratch):
    bufs = scratch[:_NBUF]
    sem_in = scratch[_NBUF:2 * _NBUF]
    sem_out = scratch[2 * _NBUF:]
    wid = lax.axis_index("s") * _NC + lax.axis_index("c")
    base = wid * _ROWS_W

    def start_in(i):
        b = i % _NBUF
        cp = pltpu.make_async_copy(
            x_hbm.at[pl.ds(base + i * _CH, _CH)], bufs[b], sem_in[b])
        cp.start()
        return cp

    def start_out(i):
        b = i % _NBUF
        cp = pltpu.make_async_copy(
            bufs[b], out_hbm.at[pl.ds(base + i * _CH, _CH)], sem_out[b])
        cp.start()
        return cp

    in_cp = [None] * _NCHUNK
    out_cp = [None] * _NCHUNK
    for i in range(_NBUF):
        in_cp[i] = start_in(i)
    for i in range(_NCHUNK):
        nxt = i + _NBUF
        in_cp[i].wait()
        out_cp[i] = start_out(i)
        if nxt < _NCHUNK:
            out_cp[nxt - _NBUF].wait()
            in_cp[nxt] = start_in(nxt)
    for j in range(max(0, _NCHUNK - _NBUF), _NCHUNK):
        out_cp[j].wait()


@jax.jit
def kernel(inputs):
    mesh = plsc.VectorSubcoreMesh(
        core_axis_name="c", subcore_axis_name="s",
        num_cores=_NC, num_subcores=_NS)
    routed_flat = pl.kernel(
        _dispatch_body,
        out_type=jax.ShapeDtypeStruct((_T, _D), jnp.float32),
        mesh=mesh,
        scratch_types=(
            [pltpu.VMEM((_CH, _D), jnp.float32) for _ in range(_NBUF)]
            + [pltpu.SemaphoreType.DMA for _ in range(2 * _NBUF)]
        ),
    )(inputs)
    return routed_flat.reshape(_PATHS, _T // _PATHS, _D)


# speedup vs baseline: 1.0539x; 1.0539x over previous
import jax
import jax.numpy as jnp
from jax import lax
from jax.experimental import pallas as pl
from jax.experimental.pallas import tpu as pltpu
from jax.experimental.pallas import tpu_sc as plsc

_PATHS = 64
_T = 32768
_D = 768
_NC = 2
_NS = 16
_NW = _NC * _NS
_ROWS_W = _T // _NW
_CH = 64
_NCHUNK = _ROWS_W // _CH
_NBUF = 2


def _dispatch_body(x_hbm, out_hbm, shared, *sems):
    sem_in = sems[:_NBUF]
    sem_out = sems[_NBUF:]
    cid = lax.axis_index("c")
    sid = lax.axis_index("s")
    wid = sid * _NC + cid
    base = wid * _ROWS_W

    def buf(i):
        b = i % _NBUF
        return shared.at[pl.ds((sid * _NBUF + b) * _CH, _CH)]

    def start_in(i):
        cp = pltpu.make_async_copy(
            x_hbm.at[pl.ds(base + i * _CH, _CH)], buf(i), sem_in[i % _NBUF])
        cp.start()
        return cp

    def start_out(i):
        cp = pltpu.make_async_copy(
            buf(i), out_hbm.at[pl.ds(base + i * _CH, _CH)], sem_out[i % _NBUF])
        cp.start()
        return cp

    in_cp = [None] * _NCHUNK
    out_cp = [None] * _NCHUNK
    in_cp[0] = start_in(0)
    for i in range(_NCHUNK):
        nxt = i + 1
        if nxt < _NCHUNK:
            if nxt >= _NBUF:
                out_cp[nxt - _NBUF].wait()
            in_cp[nxt] = start_in(nxt)
        in_cp[i].wait()
        out_cp[i] = start_out(i)
    for j in range(max(0, _NCHUNK - _NBUF), _NCHUNK):
        out_cp[j].wait()


@jax.jit
def kernel(inputs):
    mesh = plsc.VectorSubcoreMesh(
        core_axis_name="c", subcore_axis_name="s",
        num_cores=_NC, num_subcores=_NS)
    routed_flat = pl.kernel(
        _dispatch_body,
        out_type=jax.ShapeDtypeStruct((_T, _D), jnp.float32),
        mesh=mesh,
        scratch_types=(
            [pltpu.VMEM_SHARED((_NS * _NBUF * _CH, _D), jnp.float32)]
            + [pltpu.SemaphoreType.DMA for _ in range(2 * _NBUF)]
        ),
    )(inputs)
    return routed_flat.reshape(_PATHS, _T // _PATHS, _D)
